# baseline (speedup 1.0000x reference)
import jax
import jax.numpy as jnp
from jax import lax
from jax.experimental import pallas as pl

N = 4096
DIM = 6
NSP = 3
R = 0.5
HID = 128
MO = 64
NB = 8
DP = 8
BI = 1024
NBLK = N // BI


def _dot_t(a, b, precision=None):
    return lax.dot_general(a, b, (((1,), (1,)), ((), ())),
                           preferred_element_type=jnp.float32,
                           precision=precision)


def _count_body(cur_i, cur_j, bcol_i, brow_j, c1_ref):
    jb = pl.program_id(1)

    @pl.when(jb == 0)
    def _():
        c1_ref[...] = jnp.zeros((BI, 1), jnp.float32)

    bc_i = bcol_i[...]
    b_j = brow_j[...]
    overlap = ((jnp.min(b_j) <= jnp.max(bc_i))
               & (jnp.max(b_j) >= jnp.min(bc_i)))

    @pl.when(overlap)
    def _():
        col = lax.broadcasted_iota(jnp.int32, (BI, DP), 1)
        pos_i = jnp.where(col < NSP, cur_i[...], 0.0)
        pos_j = jnp.where(col < NSP, cur_j[...], 0.0)
        sq_i = jnp.sum(pos_i * pos_i, axis=1, keepdims=True)
        sq_j = _dot_t(jnp.ones((1, DP), jnp.float32), pos_j * pos_j,
                      precision=lax.Precision.HIGHEST)
        dotmat = _dot_t(pos_i.astype(jnp.bfloat16), pos_j.astype(jnp.bfloat16))
        thr = ((sq_i - R * R) * 0.5) + (sq_j * 0.5)
        m = (dotmat > thr) & (bc_i == b_j)
        mf = jnp.where(m, 1.0, 0.0).astype(jnp.bfloat16)
        c1_ref[...] += jnp.dot(mf, jnp.ones((BI, 1), jnp.bfloat16),
                               preferred_element_type=jnp.float32)


def _upd_body(cur_ref, c1_ref, w1p, b1r, w2, wor, out_ref):
    x = cur_ref[...]
    pre1 = jnp.dot(x.astype(jnp.bfloat16), w1p[...].astype(jnp.bfloat16),
                   preferred_element_type=jnp.float32) + b1r[...]
    wo_f = wor[...].astype(jnp.bfloat16).astype(jnp.float32)
    c = c1_ref[...] - 1.0
    dmsg = c * wo_f
    dh = _dot_t(dmsg.astype(jnp.bfloat16), w2[...].astype(jnp.bfloat16))
    dpre = jnp.where(pre1 > 0, dh, 0.0)
    dx = _dot_t(dpre.astype(jnp.bfloat16), w1p[...].astype(jnp.bfloat16))
    out_ref[...] = x - dx * 0.1


@jax.jit
def _one_step(cur_pad, bcol, brow, w1p, b1r, w2, wor):
    c1 = pl.pallas_call(
        _count_body,
        grid=(NBLK, NBLK),
        in_specs=[
            pl.BlockSpec((BI, DP), lambda i, j: (i, 0)),
            pl.BlockSpec((BI, DP), lambda i, j: (j, 0)),
            pl.BlockSpec((BI, 1), lambda i, j: (i, 0)),
            pl.BlockSpec((1, BI), lambda i, j: (0, j)),
        ],
        out_specs=pl.BlockSpec((BI, 1), lambda i, j: (i, 0)),
        out_shape=jax.ShapeDtypeStruct((N, 1), jnp.float32),
    )(cur_pad, cur_pad, bcol, brow)
    return pl.pallas_call(
        _upd_body,
        out_shape=jax.ShapeDtypeStruct((N, DP), jnp.float32),
    )(cur_pad, c1, w1p, b1r, w2, wor)


def kernel(x, batch, steps, W1, b1, W2, b2, Wo, bo):
    cur_pad = jnp.pad(x, ((0, 0), (0, DP - DIM)))
    bcol = batch.reshape(N, 1)
    brow = batch.reshape(1, N)
    w1p = jnp.pad(W1, ((0, DP - DIM), (0, 0)))
    b1r = b1.reshape(1, HID)
    wor = Wo.reshape(1, MO)

    def step(_, cp):
        return _one_step(cp, bcol, brow, w1p, b1r, W2, wor)

    out = lax.fori_loop(0, steps, step, cur_pad)
    return out[:, :DIM]
